# baseline (speedup 1.0000x reference)
import functools

import jax
import jax.numpy as jnp
from jax import lax
from jax.experimental import pallas as pl
from jax.experimental.pallas import tpu as pltpu
from jax.experimental.pallas import tpu_sc as plsc

B = 16384
D = 64
CH = 128

_info = plsc.get_sparse_core_info()
NC, NS, L = _info.num_cores, _info.num_subcores, _info.num_lanes
NW = NC * NS
BPW = B // NW
NCHUNK = BPW // CH


def _body(uid, wro, cit, coa, ven, aff,
          author_t, venue_t, affil_t, rel_t, doc_t,
          o_user, o_wrote, o_cited, o_coauth, o_venue, o_affil,
          o_r0, o_r1, o_r2, o_r3, o_r4,
          idx_v, rows_v, rel_rows, rel_idx, sem):
    wid = lax.axis_index("s") * NC + lax.axis_index("c")
    base = wid * BPW

    gathers = (
        (author_t, uid, o_user),
        (doc_t, wro, o_wrote),
        (doc_t, cit, o_cited),
        (author_t, coa, o_coauth),
        (venue_t, ven, o_venue),
        (affil_t, aff, o_affil),
    )
    for table, idx_hbm, out in gathers:
        pltpu.sync_copy(idx_hbm.at[wid], idx_v)
        cps = [
            pltpu.async_copy(table.at[idx_v.at[j]],
                             rows_v.at[pl.ds(j * CH, CH)], sem)
            for j in range(NCHUNK)
        ]
        for c in cps:
            c.wait()
        pltpu.sync_copy(rows_v, out.at[pl.ds(base, BPW)])

    rel_outs = (o_r0, o_r1, o_r2, o_r3, o_r4)
    for r, out in enumerate(rel_outs):
        for i in range(CH // L):
            rel_idx[pl.ds(i * L, L)] = jnp.full((L,), r, jnp.int32)
        pltpu.async_copy(rel_t.at[rel_idx], rel_rows, sem).wait()
        for j in range(NCHUNK):
            pltpu.sync_copy(rel_rows, out.at[pl.ds(base + j * CH, CH)])


@jax.jit
def _run(uid, wro, cit, coa, ven, aff, author_t, venue_t, affil_t, rel_t, doc_t):
    out = jax.ShapeDtypeStruct((B, D), jnp.float32)
    k = pl.kernel(
        _body,
        out_type=[out] * 11,
        mesh=plsc.VectorSubcoreMesh(core_axis_name="c", subcore_axis_name="s"),
        scratch_types=[
            pltpu.VMEM((NCHUNK, CH), jnp.int32),
            pltpu.VMEM((BPW, D), jnp.float32),
            pltpu.VMEM((CH, D), jnp.float32),
            pltpu.VMEM((CH,), jnp.int32),
            pltpu.SemaphoreType.DMA,
        ],
        compiler_params=pltpu.CompilerParams(use_tc_tiling_on_sc=False),
    )
    return tuple(k(uid, wro, cit, coa, ven, aff, author_t, venue_t, affil_t, rel_t, doc_t))


def kernel(user_id, wrote, cited, coauthor, venue, affiliation,
           author_table, venue_table, affiliation_table, relation_table,
           doc_embs):
    def prep(i):
        return i.astype(jnp.int32).reshape(NW, NCHUNK, CH)

    return _run(prep(user_id), prep(wrote), prep(cited), prep(coauthor),
                prep(venue), prep(affiliation),
                author_table, venue_table, affiliation_table, relation_table,
                doc_embs)
